# baseline (speedup 1.0000x reference)
import jax
import jax.numpy as jnp
from jax import lax
from jax.experimental import pallas as pl
from jax.experimental.pallas import tpu as pltpu
from jax.experimental.pallas import tpu_sc as plsc

_B = 4096
_D = 32
_NC, _NS = 2, 16
_NW = _NC * _NS
_BPW = _B // _NW
_LANES = 16


def _sc_gather_body(uids, iids, mfu, mfi, mlu, mli,
                    out_u, out_i, out_mf,
                    idx_u, idx_i, buf_u, buf_i, buf_a, buf_b, sem):
    wid = lax.axis_index("s") * _NC + lax.axis_index("c")
    base = wid * _BPW
    pltpu.sync_copy(uids.at[pl.ds(base, _BPW)], idx_u)
    pltpu.sync_copy(iids.at[pl.ds(base, _BPW)], idx_i)
    c1 = pltpu.async_copy(mlu.at[idx_u], buf_u, sem)
    c2 = pltpu.async_copy(mli.at[idx_i], buf_i, sem)
    c3 = pltpu.async_copy(mfu.at[idx_u], buf_a, sem)
    c4 = pltpu.async_copy(mfi.at[idx_i], buf_b, sem)
    c1.wait()
    c2.wait()
    pltpu.sync_copy(buf_u, out_u.at[pl.ds(base, _BPW)])
    pltpu.sync_copy(buf_i, out_i.at[pl.ds(base, _BPW)])
    c3.wait()
    c4.wait()

    def _row(r, carry):
        for c in range(_D // _LANES):
            s = pl.ds(c * _LANES, _LANES)
            buf_a[r, s] = buf_a[r, s] * buf_b[r, s]
        return carry

    lax.fori_loop(0, _BPW, _row, 0)
    pltpu.sync_copy(buf_a, out_mf.at[pl.ds(base, _BPW)])


@jax.jit
def _sc_gather(uids, iids, mfu, mfi, mlu, mli):
    mesh = plsc.VectorSubcoreMesh(
        core_axis_name="c", subcore_axis_name="s",
        num_cores=_NC, num_subcores=_NS)
    f32 = jnp.float32
    return pl.kernel(
        _sc_gather_body,
        out_type=[jax.ShapeDtypeStruct((_B, _D), f32)] * 3,
        mesh=mesh,
        scratch_types=[
            pltpu.VMEM((_BPW,), jnp.int32),
            pltpu.VMEM((_BPW,), jnp.int32),
            pltpu.VMEM((_BPW, _D), f32),
            pltpu.VMEM((_BPW, _D), f32),
            pltpu.VMEM((_BPW, _D), f32),
            pltpu.VMEM((_BPW, _D), f32),
            pltpu.SemaphoreType.DMA,
        ],
        compiler_params=pltpu.CompilerParams(use_tc_tiling_on_sc=False),
    )(uids, iids, mfu, mfi, mlu, mli)


def _tc_mlp_body(u_ref, i_ref, mf_ref, w1u_ref, w1i_ref, b1_ref,
                 w2_ref, b2_ref, w3_ref, b3_ref,
                 wp_mf_ref, wp_mlp_ref, bp_ref, out_ref):
    h = jnp.maximum(
        jnp.dot(u_ref[...], w1u_ref[...], preferred_element_type=jnp.float32)
        + jnp.dot(i_ref[...], w1i_ref[...], preferred_element_type=jnp.float32)
        + b1_ref[...][None, :], 0.0)
    h = jnp.maximum(
        jnp.dot(h, w2_ref[...], preferred_element_type=jnp.float32)
        + b2_ref[...][None, :], 0.0)
    h = jnp.maximum(
        jnp.dot(h, w3_ref[...], preferred_element_type=jnp.float32)
        + b3_ref[...][None, :], 0.0)
    logit = (jnp.sum(mf_ref[...] * wp_mf_ref[...][None, :], axis=1, keepdims=True)
             + jnp.sum(h * wp_mlp_ref[...][None, :], axis=1, keepdims=True)
             + bp_ref[...][None, :])
    out_ref[...] = jax.nn.sigmoid(logit)


@jax.jit
def _tc_mlp(u, i, mf, w1u, w1i, b1, w2, b2, w3, b3, wp_mf, wp_mlp, bp):
    return pl.pallas_call(
        _tc_mlp_body,
        out_shape=jax.ShapeDtypeStruct((_B, 1), jnp.float32),
    )(u, i, mf, w1u, w1i, b1, w2, b2, w3, b3, wp_mf, wp_mlp, bp)


def kernel(user_ids, item_ids, mf_user_table, mf_item_table,
           mlp_user_table, mlp_item_table, W1, b1, W2, b2, W3, b3, Wp, bp):
    uids = user_ids.astype(jnp.int32)
    iids = item_ids.astype(jnp.int32)
    mlp_u, mlp_i, mf_prod = _sc_gather(
        uids, iids, mf_user_table, mf_item_table,
        mlp_user_table, mlp_item_table)
    return _tc_mlp(
        mlp_u, mlp_i, mf_prod,
        W1[:_D, :], W1[_D:, :], b1, W2, b2, W3, b3,
        Wp[:_D, 0], Wp[_D:, 0], bp)
